# baseline (speedup 1.0000x reference)
import functools

import jax
import jax.numpy as jnp
from jax import lax
from jax.experimental import pallas as pl
from jax.experimental.pallas import tpu as pltpu
from jax.experimental.pallas import tpu_sc as plsc

B = 16384
C = 10
NC = 2
NS = 16
NW = NC * NS
ROWS_PER_W = B // NW
CHUNK = ROWS_PER_W * C
GROUPS = ROWS_PER_W // 16

_EPS = 1e-7
_ONE_M_EPS = 1.0 - 1e-7
_LN2 = 0.6931471805599453
_SQRT2 = 1.4142135


def _neg_log_f32(p):
    bits = lax.bitcast_convert_type(p, jnp.int32)
    e = lax.shift_right_logical(bits, 23) - 127
    m = lax.bitcast_convert_type(
        (bits & jnp.int32(0x007FFFFF)) | jnp.int32(0x3F800000), jnp.float32)
    big = m > jnp.float32(_SQRT2)
    m = jnp.where(big, m * jnp.float32(0.5), m)
    ef = e.astype(jnp.float32) + jnp.where(big, jnp.float32(1.0), jnp.float32(0.0))
    z = (m - jnp.float32(1.0)) / (m + jnp.float32(1.0))
    z2 = z * z
    poly = z * (jnp.float32(2.0) + z2 * (jnp.float32(2.0 / 3.0)
               + z2 * (jnp.float32(2.0 / 5.0) + z2 * jnp.float32(2.0 / 7.0))))
    return -(ef * jnp.float32(_LN2) + poly)


def _sc_loss_body(soft_hbm, tgt_hbm, out_hbm, soft_v, tgt_v, acc_v, red_v, shared):
    cid = lax.axis_index("c")
    sid = lax.axis_index("s")
    wid = sid * NC + cid
    base = wid * CHUNK
    pltpu.sync_copy(soft_hbm.at[pl.ds(base, CHUNK)], soft_v)
    pltpu.sync_copy(tgt_hbm.at[pl.ds(base, CHUNK)], tgt_v)

    lane_row0 = lax.broadcasted_iota(jnp.int32, (16,), 0) * C

    def group(j, acc):
        idx0 = j * (16 * C) + lane_row0
        ssum = jnp.zeros((16,), jnp.float32)
        smax = jnp.full((16,), -jnp.inf, jnp.float32)
        sel = jnp.zeros((16,), jnp.float32)
        for cc in range(C):
            idx = idx0 + cc
            t = plsc.load_gather(tgt_v, [idx])
            sv = plsc.load_gather(soft_v, [idx])
            ssum = ssum + sv
            hit = t > smax
            smax = jnp.where(hit, t, smax)
            sel = jnp.where(hit, sv, sel)
        p = sel / ssum
        p = jnp.minimum(jnp.maximum(p, jnp.float32(_EPS)), jnp.float32(_ONE_M_EPS))
        return acc + _neg_log_f32(p)

    acc = lax.fori_loop(0, GROUPS, group, jnp.zeros((16,), jnp.float32))
    acc_v[...] = acc
    pltpu.sync_copy(acc_v, shared.at[sid])
    plsc.subcore_barrier()

    @pl.when(sid == 0)
    def _reduce():
        pltpu.sync_copy(shared, red_v)
        tot = red_v[0, :]
        for i in range(1, NS):
            tot = tot + red_v[i, :]
        total = jnp.sum(tot) * jnp.float32(0.5 / B)
        acc_v[...] = jnp.broadcast_to(total, (16,))
        pltpu.sync_copy(acc_v, out_hbm.at[cid])


@functools.cache
def _sc_loss():
    return pl.kernel(
        _sc_loss_body,
        mesh=plsc.VectorSubcoreMesh(core_axis_name="c", subcore_axis_name="s"),
        compiler_params=pltpu.CompilerParams(
            needs_layout_passes=False, use_tc_tiling_on_sc=False),
        out_type=jax.ShapeDtypeStruct((NC, 16), jnp.float32),
        scratch_types=[
            pltpu.VMEM((CHUNK,), jnp.float32),
            pltpu.VMEM((CHUNK,), jnp.float32),
            pltpu.VMEM((16,), jnp.float32),
            pltpu.VMEM((NS, 16), jnp.float32),
            pltpu.VMEM_SHARED((NS, 16), jnp.float32),
        ],
    )


def kernel(softmax_output, ef_out, targets):
    partials = _sc_loss()(softmax_output.reshape(-1), targets.reshape(-1))
    loss = partials[0, 0] + partials[1, 0]
    return (softmax_output, ef_out, loss)


# speedup vs baseline: 1.0088x; 1.0088x over previous
import functools

import jax
import jax.numpy as jnp
from jax import lax
from jax.experimental import pallas as pl
from jax.experimental.pallas import tpu as pltpu
from jax.experimental.pallas import tpu_sc as plsc

B = 16384
C = 10
NC = 2
NS = 16
NW = NC * NS
ROWS_PER_W = B // NW
CHUNK = ROWS_PER_W * C
GROUPS = ROWS_PER_W // 16

_EPS = 1e-7
_ONE_M_EPS = 1.0 - 1e-7
_LN2 = 0.6931471805599453
_SQRT2 = 1.4142135


def _neg_log_f32(p):
    bits = lax.bitcast_convert_type(p, jnp.int32)
    e = lax.shift_right_logical(bits, 23) - 127
    m = lax.bitcast_convert_type(
        (bits & jnp.int32(0x007FFFFF)) | jnp.int32(0x3F800000), jnp.float32)
    big = m > jnp.float32(_SQRT2)
    m = jnp.where(big, m * jnp.float32(0.5), m)
    ef = e.astype(jnp.float32) + jnp.where(big, jnp.float32(1.0), jnp.float32(0.0))
    z = (m - jnp.float32(1.0)) / (m + jnp.float32(1.0))
    z2 = z * z
    poly = z * (jnp.float32(2.0) + z2 * (jnp.float32(2.0 / 3.0)
               + z2 * (jnp.float32(2.0 / 5.0) + z2 * jnp.float32(2.0 / 7.0))))
    return -(ef * jnp.float32(_LN2) + poly)


def _sc_loss_body(soft_hbm, tgt_hbm, out_hbm, soft_v, tgt_v, acc_v, red_v, shared):
    cid = lax.axis_index("c")
    sid = lax.axis_index("s")
    wid = sid * NC + cid
    base = wid * CHUNK
    pltpu.sync_copy(soft_hbm.at[pl.ds(base, CHUNK)], soft_v)
    pltpu.sync_copy(tgt_hbm.at[pl.ds(base, CHUNK)], tgt_v)

    lane_row0 = lax.broadcasted_iota(jnp.int32, (16,), 0) * C

    def group(j, acc):
        idx0 = j * (16 * C) + lane_row0
        ssum = jnp.zeros((16,), jnp.float32)
        smax = jnp.full((16,), -jnp.inf, jnp.float32)
        sel = jnp.zeros((16,), jnp.float32)
        for cc in range(C):
            idx = idx0 + cc
            t = plsc.load_gather(tgt_v, [idx])
            sv = plsc.load_gather(soft_v, [idx])
            ssum = ssum + sv
            hit = t > smax
            smax = jnp.where(hit, t, smax)
            sel = jnp.where(hit, sv, sel)
        p = sel / ssum
        p = jnp.minimum(jnp.maximum(p, jnp.float32(_EPS)), jnp.float32(_ONE_M_EPS))
        return acc + _neg_log_f32(p)

    acc = lax.fori_loop(0, GROUPS, group, jnp.zeros((16,), jnp.float32))
    acc_v[...] = acc
    pltpu.sync_copy(acc_v, shared.at[sid])
    plsc.subcore_barrier()

    @pl.when(sid == 0)
    def _reduce():
        pltpu.sync_copy(shared, red_v)
        tot = red_v[0, :]
        for i in range(1, NS):
            tot = tot + red_v[i, :]
        total = jnp.sum(tot) * jnp.float32(0.5 / B)
        acc_v[...] = jnp.broadcast_to(total, (16,))
        pltpu.sync_copy(acc_v, out_hbm.at[cid])


@functools.cache
def _sc_loss():
    return pl.kernel(
        _sc_loss_body,
        mesh=plsc.VectorSubcoreMesh(core_axis_name="c", subcore_axis_name="s"),
        compiler_params=pltpu.CompilerParams(
            needs_layout_passes=False,
            use_tc_tiling_on_sc=False,
            skip_device_barrier=True,
            disable_bounds_checks=True,
            disable_semaphore_checks=True,
        ),
        out_type=jax.ShapeDtypeStruct((NC, 16), jnp.float32),
        scratch_types=[
            pltpu.VMEM((CHUNK,), jnp.float32),
            pltpu.VMEM((CHUNK,), jnp.float32),
            pltpu.VMEM((16,), jnp.float32),
            pltpu.VMEM((NS, 16), jnp.float32),
            pltpu.VMEM_SHARED((NS, 16), jnp.float32),
        ],
    )


def kernel(softmax_output, ef_out, targets):
    partials = _sc_loss()(softmax_output.reshape(-1), targets.reshape(-1))
    loss = partials[0, 0] + partials[1, 0]
    return (softmax_output, ef_out, loss)
